# baseline (speedup 1.0000x reference)
import jax
import jax.numpy as jnp
from jax import lax
from jax.experimental import pallas as pl
from jax.experimental.pallas import tpu as pltpu
from jax.experimental.pallas import tpu_sc as plsc

_VOCAB = 1000000
_EMBED = 64
_BATCH = 4096
_SEQ = 200
_SEQP = 208
_NC = 2
_NS = 16
_NW = _NC * _NS
_ROWS_PER_W = _BATCH // _NW
_IDX_PER_W = _ROWS_PER_W * _SEQP
_TC_BLK = 8192


def _tw_body(tbl_ref, w_ref, o_ref):
    w = w_ref[...][:, 0]
    o_ref[...] = jnp.sum(tbl_ref[...] * w[None, :], axis=1)


def _table_times_w(table, W):
    return pl.pallas_call(
        _tw_body,
        grid=(_VOCAB // _TC_BLK,),
        in_specs=[
            pl.BlockSpec((_TC_BLK, _EMBED), lambda i: (i, 0)),
            pl.BlockSpec((_EMBED, 1), lambda i: (0, 0)),
        ],
        out_specs=pl.BlockSpec((_TC_BLK,), lambda i: (i,)),
        out_shape=jax.ShapeDtypeStruct((_VOCAB,), jnp.float32),
    )(table, W)


def _sc_body(tw_hbm, xt_hbm, b_hbm, out_hbm, idx_v, vals_v, res_v, b_v, sem):
    wid = lax.axis_index("s") * _NC + lax.axis_index("c")
    base = wid * _ROWS_PER_W

    pltpu.sync_copy(xt_hbm.at[pl.ds(wid * _IDX_PER_W, _IDX_PER_W)], idx_v)
    pltpu.sync_copy(b_hbm, b_v)
    bvec = b_v[...]
    pltpu.async_copy(tw_hbm.at[idx_v], vals_v, sem).wait()

    zero16 = jnp.zeros((16,), jnp.float32)
    for sub in range(_ROWS_PER_W // 16):
        def cbody(c, carry):
            s, cnt = carry
            off = c * _ROWS_PER_W + sub * 16
            v = vals_v[pl.ds(off, 16)]
            ii = idx_v[pl.ds(off, 16)]
            s = s + v
            cnt = cnt + jnp.minimum(ii, 1).astype(jnp.float32)
            return (s, cnt)

        s, cnt = lax.fori_loop(0, _SEQP, cbody, (zero16, zero16))
        z = s / (cnt + 1e-8) + bvec
        res_v[pl.ds(sub * 16, 16)] = 1.0 / (1.0 + jnp.exp(-z))

    pltpu.sync_copy(res_v, out_hbm.at[pl.ds(base, _ROWS_PER_W)])


def _sc_call():
    return pl.kernel(
        _sc_body,
        out_type=jax.ShapeDtypeStruct((_BATCH,), jnp.float32),
        mesh=plsc.VectorSubcoreMesh(
            core_axis_name="c", subcore_axis_name="s",
            num_cores=_NC, num_subcores=_NS),
        scratch_types=[
            pltpu.VMEM((_IDX_PER_W,), jnp.int32),
            pltpu.VMEM((_IDX_PER_W,), jnp.float32),
            pltpu.VMEM((_ROWS_PER_W,), jnp.float32),
            pltpu.VMEM((16,), jnp.float32),
            pltpu.SemaphoreType.DMA,
        ],
    )


def kernel(x, lengths, table, W, b):
    del lengths
    tw = _table_times_w(table, W)
    xp = jnp.concatenate(
        [x, jnp.zeros((_BATCH, _SEQP - _SEQ), jnp.int32)], axis=1
    )
    xt = xp.reshape(_NW, _ROWS_PER_W, _SEQP).transpose(0, 2, 1).reshape(-1)
    b16 = jnp.broadcast_to(b.reshape(()), (16,))
    return _sc_call()(tw, xt, b16)
